# baseline (speedup 1.0000x reference)
import functools

import jax
import jax.numpy as jnp
from jax import lax
from jax.experimental import pallas as pl
from jax.experimental.pallas import tpu as pltpu
from jax.experimental.pallas import tpu_sc as plsc


@functools.cache
def _make_sc_gather(BS, V, D, NC, NS):
    NW = NC * NS
    RPW = BS // NW
    K = 32
    NCH = RPW // K
    mesh = plsc.VectorSubcoreMesh(core_axis_name="c", subcore_axis_name="s")

    @functools.partial(
        pl.kernel,
        mesh=mesh,
        out_type=jax.ShapeDtypeStruct((BS, D), jnp.float32),
        scratch_types=[
            pltpu.VMEM((NCH, K), jnp.int32),
            pltpu.VMEM((K, D), jnp.float32),
            pltpu.VMEM((K, D), jnp.float32),
            pltpu.SemaphoreType.DMA,
            pltpu.SemaphoreType.DMA,
            pltpu.SemaphoreType.DMA,
            pltpu.SemaphoreType.DMA,
        ],
    )
    def sc_gather(ids_hbm, table_hbm, out_hbm,
                  idx_v, tok0_v, tok1_v, gsem0, gsem1, osem0, osem1):
        tok = (tok0_v, tok1_v)
        gsem = (gsem0, gsem1)
        osem = (osem0, osem1)
        wid = lax.axis_index("s") * NC + lax.axis_index("c")
        base = wid * RPW

        pltpu.sync_copy(ids_hbm.at[wid], idx_v)

        def start_in(k, slot):
            pltpu.async_copy(table_hbm.at[idx_v.at[k]], tok[slot], gsem[slot])

        def wait_in(k, slot):
            pltpu.make_async_copy(
                table_hbm.at[idx_v.at[k]], tok[slot], gsem[slot]).wait()

        def start_out(k, slot):
            pltpu.async_copy(
                tok[slot], out_hbm.at[pl.ds(base + k * K, K)], osem[slot])

        def wait_out(k, slot):
            pltpu.make_async_copy(
                tok[slot], out_hbm.at[pl.ds(base + k * K, K)],
                osem[slot]).wait()

        def phase(k, cur, other):
            @pl.when(k >= 1)
            def _():
                wait_out(k - 1, other)

            @pl.when(k + 1 < NCH)
            def _():
                start_in(k + 1, other)

            wait_in(k, cur)
            start_out(k, cur)

        start_in(0, 0)

        def pair_body(c2, carry):
            phase(2 * c2, 0, 1)
            phase(2 * c2 + 1, 1, 0)
            return carry

        lax.fori_loop(0, NCH // 2, pair_body, 0)
        wait_out(NCH - 1, 1)

    return sc_gather


def _tc_ln_body(BLK, x_ref, pos_ref, g_ref, b_ref, o_ref):
    j = pl.program_id(0)
    x = x_ref[...] + pos_ref[pl.ds(j * BLK, BLK), :]
    mu = jnp.mean(x, axis=-1, keepdims=True)
    d = x - mu
    var = jnp.mean(d * d, axis=-1, keepdims=True)
    o_ref[...] = d * lax.rsqrt(var + 1e-5) * g_ref[...] + b_ref[...]


@functools.cache
def _make_tc_ln(BS, S, D):
    BLK = 256
    B = BS // S
    nsb = S // BLK

    return pl.pallas_call(
        functools.partial(_tc_ln_body, BLK),
        grid=(nsb, B),
        in_specs=[
            pl.BlockSpec((BLK, D), lambda j, b: (b * nsb + j, 0)),
            pl.BlockSpec((S, D), lambda j, b: (0, 0)),
            pl.BlockSpec((1, D), lambda j, b: (0, 0)),
            pl.BlockSpec((1, D), lambda j, b: (0, 0)),
        ],
        out_specs=pl.BlockSpec((BLK, D), lambda j, b: (b * nsb + j, 0)),
        out_shape=jax.ShapeDtypeStruct((BS, D), jnp.float32),
    )


def kernel(input_ids, token_table, pos_table, ln_gamma, ln_beta):
    B, S = input_ids.shape
    V, D = token_table.shape
    info = plsc.get_sparse_core_info()
    NC, NS = info.num_cores, info.num_subcores
    NW = NC * NS
    BS = B * S
    K = 32
    NSPLIT = 2
    HB = B // NSPLIT
    ids_flat = input_ids.astype(jnp.int32).reshape(BS)
    sc = _make_sc_gather(BS // NSPLIT, V, D, NC, NS)
    tc = _make_tc_ln(BS // NSPLIT, S, D)
    g1 = ln_gamma.reshape(1, D)
    b1 = ln_beta.reshape(1, D)
    halves = []
    for h in range(NSPLIT):
        idsh = lax.slice(ids_flat, (h * BS // NSPLIT,),
                         ((h + 1) * BS // NSPLIT,))
        ids3 = idsh.reshape(NW, BS // NSPLIT // NW // K, K)
        gathered = sc(ids3, token_table)
        halves.append(tc(gathered, pos_table, g1, b1))
    return jnp.concatenate(halves, axis=0).reshape(B, S, D)


# speedup vs baseline: 1.2486x; 1.2486x over previous
import functools

import jax
import jax.numpy as jnp
from jax import lax
from jax.experimental import pallas as pl
from jax.experimental.pallas import tpu as pltpu
from jax.experimental.pallas import tpu_sc as plsc


@functools.cache
def _make_sc_gather(BS, V, D, NC, NS):
    NW = NC * NS
    RPW = BS // NW
    K = 32
    NCH = RPW // K
    mesh = plsc.VectorSubcoreMesh(core_axis_name="c", subcore_axis_name="s")

    @functools.partial(
        pl.kernel,
        mesh=mesh,
        out_type=jax.ShapeDtypeStruct((BS, D), jnp.float32),
        scratch_types=[
            pltpu.VMEM((NCH, K), jnp.int32),
            pltpu.VMEM((K, D), jnp.float32),
            pltpu.VMEM((K, D), jnp.float32),
            pltpu.SemaphoreType.DMA,
            pltpu.SemaphoreType.DMA,
            pltpu.SemaphoreType.DMA,
            pltpu.SemaphoreType.DMA,
        ],
    )
    def sc_gather(ids_hbm, table_hbm, out_hbm,
                  idx_v, tok0_v, tok1_v, gsem0, gsem1, osem0, osem1):
        tok = (tok0_v, tok1_v)
        gsem = (gsem0, gsem1)
        osem = (osem0, osem1)
        wid = lax.axis_index("s") * NC + lax.axis_index("c")
        base = wid * RPW

        pltpu.sync_copy(ids_hbm.at[wid], idx_v)

        def start_in(k, slot):
            pltpu.async_copy(table_hbm.at[idx_v.at[k]], tok[slot], gsem[slot])

        def wait_in(k, slot):
            pltpu.make_async_copy(
                table_hbm.at[idx_v.at[k]], tok[slot], gsem[slot]).wait()

        def start_out(k, slot):
            pltpu.async_copy(
                tok[slot], out_hbm.at[pl.ds(base + k * K, K)], osem[slot])

        def wait_out(k, slot):
            pltpu.make_async_copy(
                tok[slot], out_hbm.at[pl.ds(base + k * K, K)],
                osem[slot]).wait()

        def phase(k, cur, other):
            @pl.when(k >= 1)
            def _():
                wait_out(k - 1, other)

            @pl.when(k + 1 < NCH)
            def _():
                start_in(k + 1, other)

            wait_in(k, cur)
            start_out(k, cur)

        start_in(0, 0)

        def pair_body(c2, carry):
            phase(2 * c2, 0, 1)
            phase(2 * c2 + 1, 1, 0)
            return carry

        lax.fori_loop(0, NCH // 2, pair_body, 0)
        wait_out(NCH - 1, 1)

    return sc_gather


def _tc_ln_body(BLK, x_ref, pos_ref, g_ref, b_ref, o_ref):
    j = pl.program_id(0)
    x = x_ref[...] + pos_ref[pl.ds(j * BLK, BLK), :]
    mu = jnp.mean(x, axis=-1, keepdims=True)
    d = x - mu
    var = jnp.mean(d * d, axis=-1, keepdims=True)
    o_ref[...] = d * lax.rsqrt(var + 1e-5) * g_ref[...] + b_ref[...]


def _tc_ln_body_acc(BLK, acc_ref, x_ref, pos_ref, g_ref, b_ref, o_ref):
    _tc_ln_body(BLK, x_ref, pos_ref, g_ref, b_ref, o_ref)


@functools.cache
def _make_tc_ln(BS, S, D, HB, hoff, aliased):
    BLK = 256
    nsb = S // BLK

    x_spec = pl.BlockSpec((BLK, D), lambda j, b: (b * nsb + j, 0))
    pos_spec = pl.BlockSpec((S, D), lambda j, b: (0, 0))
    vec_spec = pl.BlockSpec((1, D), lambda j, b: (0, 0))
    out_spec = pl.BlockSpec(
        (BLK, D), lambda j, b: ((hoff + b) * nsb + j, 0))
    if aliased:
        dummy = pl.BlockSpec((8, 128), lambda j, b: (0, 0))
        return pl.pallas_call(
            functools.partial(_tc_ln_body_acc, BLK),
            grid=(nsb, HB),
            in_specs=[dummy, x_spec, pos_spec, vec_spec, vec_spec],
            out_specs=out_spec,
            out_shape=jax.ShapeDtypeStruct((BS, D), jnp.float32),
            input_output_aliases={0: 0},
        )
    return pl.pallas_call(
        functools.partial(_tc_ln_body, BLK),
        grid=(nsb, HB),
        in_specs=[x_spec, pos_spec, vec_spec, vec_spec],
        out_specs=out_spec,
        out_shape=jax.ShapeDtypeStruct((BS, D), jnp.float32),
    )


def kernel(input_ids, token_table, pos_table, ln_gamma, ln_beta):
    B, S = input_ids.shape
    V, D = token_table.shape
    info = plsc.get_sparse_core_info()
    NC, NS = info.num_cores, info.num_subcores
    NW = NC * NS
    BS = B * S
    K = 32
    NSPLIT = 2
    HB = B // NSPLIT
    HR = BS // NSPLIT
    ids_flat = input_ids.astype(jnp.int32).reshape(BS)
    sc = _make_sc_gather(HR, V, D, NC, NS)
    g1 = ln_gamma.reshape(1, D)
    b1 = ln_beta.reshape(1, D)
    gathered = []
    for h in range(NSPLIT):
        idsh = lax.slice(ids_flat, (h * HR,), ((h + 1) * HR,))
        ids3 = idsh.reshape(NW, HR // NW // K, K)
        gathered.append(sc(ids3, token_table))
    out = _make_tc_ln(BS, S, D, HB, B - HB, False)(
        gathered[NSPLIT - 1], pos_table, g1, b1)
    for h in range(NSPLIT - 2, -1, -1):
        out = _make_tc_ln(BS, S, D, HB, h * HB, True)(
            out, gathered[h], pos_table, g1, b1)
    return out.reshape(B, S, D)


# speedup vs baseline: 1.3788x; 1.1043x over previous
import functools

import jax
import jax.numpy as jnp
from jax import lax
from jax.experimental import pallas as pl
from jax.experimental.pallas import tpu as pltpu
from jax.experimental.pallas import tpu_sc as plsc


@functools.cache
def _make_sc_gather(BS, V, D, NC, NS):
    NW = NC * NS
    RPW = BS // NW
    K = 32
    NCH = RPW // K
    mesh = plsc.VectorSubcoreMesh(core_axis_name="c", subcore_axis_name="s")

    @functools.partial(
        pl.kernel,
        mesh=mesh,
        out_type=jax.ShapeDtypeStruct((BS, D), jnp.float32),
        scratch_types=[
            pltpu.VMEM((NCH, K), jnp.int32),
            pltpu.VMEM((K, D), jnp.float32),
            pltpu.VMEM((K, D), jnp.float32),
            pltpu.SemaphoreType.DMA,
            pltpu.SemaphoreType.DMA,
            pltpu.SemaphoreType.DMA,
            pltpu.SemaphoreType.DMA,
        ],
    )
    def sc_gather(ids_hbm, table_hbm, out_hbm,
                  idx_v, tok0_v, tok1_v, gsem0, gsem1, osem0, osem1):
        tok = (tok0_v, tok1_v)
        gsem = (gsem0, gsem1)
        osem = (osem0, osem1)
        wid = lax.axis_index("s") * NC + lax.axis_index("c")
        base = wid * RPW

        pltpu.sync_copy(ids_hbm.at[wid], idx_v)

        def start_in(k, slot):
            pltpu.async_copy(table_hbm.at[idx_v.at[k]], tok[slot], gsem[slot])

        def wait_in(k, slot):
            pltpu.make_async_copy(
                table_hbm.at[idx_v.at[k]], tok[slot], gsem[slot]).wait()

        def start_out(k, slot):
            pltpu.async_copy(
                tok[slot], out_hbm.at[pl.ds(base + k * K, K)], osem[slot])

        def wait_out(k, slot):
            pltpu.make_async_copy(
                tok[slot], out_hbm.at[pl.ds(base + k * K, K)],
                osem[slot]).wait()

        def phase(k, cur, other):
            @pl.when(k >= 1)
            def _():
                wait_out(k - 1, other)

            @pl.when(k + 1 < NCH)
            def _():
                start_in(k + 1, other)

            wait_in(k, cur)
            start_out(k, cur)

        start_in(0, 0)

        def pair_body(c2, carry):
            phase(2 * c2, 0, 1)
            phase(2 * c2 + 1, 1, 0)
            return carry

        lax.fori_loop(0, NCH // 2, pair_body, 0)
        wait_out(NCH - 1, 1)

    return sc_gather


def _tc_ln_body(BLK, x_ref, pos_ref, g_ref, b_ref, o_ref):
    j = pl.program_id(0)
    x = x_ref[...] + pos_ref[pl.ds(j * BLK, BLK), :]
    mu = jnp.mean(x, axis=-1, keepdims=True)
    d = x - mu
    var = jnp.mean(d * d, axis=-1, keepdims=True)
    o_ref[...] = d * lax.rsqrt(var + 1e-5) * g_ref[...] + b_ref[...]


def _tc_ln_body_acc(BLK, acc_ref, x_ref, pos_ref, g_ref, b_ref, o_ref):
    _tc_ln_body(BLK, x_ref, pos_ref, g_ref, b_ref, o_ref)


@functools.cache
def _make_tc_ln(BS, S, D, HB, hoff, aliased):
    BLK = 512
    nsb = S // BLK

    x_spec = pl.BlockSpec((BLK, D), lambda j, b: (b * nsb + j, 0))
    pos_spec = pl.BlockSpec((S, D), lambda j, b: (0, 0))
    vec_spec = pl.BlockSpec((1, D), lambda j, b: (0, 0))
    out_spec = pl.BlockSpec(
        (BLK, D), lambda j, b: ((hoff + b) * nsb + j, 0))
    if aliased:
        dummy = pl.BlockSpec((8, 128), lambda j, b: (0, 0))
        return pl.pallas_call(
            functools.partial(_tc_ln_body_acc, BLK),
            grid=(nsb, HB),
            in_specs=[dummy, x_spec, pos_spec, vec_spec, vec_spec],
            out_specs=out_spec,
            out_shape=jax.ShapeDtypeStruct((BS, D), jnp.float32),
            input_output_aliases={0: 0},
        )
    return pl.pallas_call(
        functools.partial(_tc_ln_body, BLK),
        grid=(nsb, HB),
        in_specs=[x_spec, pos_spec, vec_spec, vec_spec],
        out_specs=out_spec,
        out_shape=jax.ShapeDtypeStruct((BS, D), jnp.float32),
    )


def kernel(input_ids, token_table, pos_table, ln_gamma, ln_beta):
    B, S = input_ids.shape
    V, D = token_table.shape
    info = plsc.get_sparse_core_info()
    NC, NS = info.num_cores, info.num_subcores
    NW = NC * NS
    BS = B * S
    K = 32
    NSPLIT = 2
    HB = B // NSPLIT
    HR = BS // NSPLIT
    ids_flat = input_ids.astype(jnp.int32).reshape(BS)
    sc = _make_sc_gather(HR, V, D, NC, NS)
    g1 = ln_gamma.reshape(1, D)
    b1 = ln_beta.reshape(1, D)
    gathered = []
    for h in range(NSPLIT):
        idsh = lax.slice(ids_flat, (h * HR,), ((h + 1) * HR,))
        ids3 = idsh.reshape(NW, HR // NW // K, K)
        gathered.append(sc(ids3, token_table))
    out = _make_tc_ln(BS, S, D, HB, B - HB, False)(
        gathered[NSPLIT - 1], pos_table, g1, b1)
    for h in range(NSPLIT - 2, -1, -1):
        out = _make_tc_ln(BS, S, D, HB, h * HB, True)(
            out, gathered[h], pos_table, g1, b1)
    return out.reshape(B, S, D)


# speedup vs baseline: 1.3935x; 1.0107x over previous
import functools

import jax
import jax.numpy as jnp
from jax import lax
from jax.experimental import pallas as pl
from jax.experimental.pallas import tpu as pltpu
from jax.experimental.pallas import tpu_sc as plsc


@functools.cache
def _make_sc_gather(BS, V, D, NC, NS):
    NW = NC * NS
    RPW = BS // NW
    K = 32
    NCH = RPW // K
    mesh = plsc.VectorSubcoreMesh(core_axis_name="c", subcore_axis_name="s")

    @functools.partial(
        pl.kernel,
        mesh=mesh,
        out_type=jax.ShapeDtypeStruct((BS, D), jnp.float32),
        scratch_types=[
            pltpu.VMEM((NCH, K), jnp.int32),
            pltpu.VMEM((K, D), jnp.float32),
            pltpu.VMEM((K, D), jnp.float32),
            pltpu.SemaphoreType.DMA,
            pltpu.SemaphoreType.DMA,
            pltpu.SemaphoreType.DMA,
            pltpu.SemaphoreType.DMA,
        ],
    )
    def sc_gather(ids_hbm, table_hbm, out_hbm,
                  idx_v, tok0_v, tok1_v, gsem0, gsem1, osem0, osem1):
        tok = (tok0_v, tok1_v)
        gsem = (gsem0, gsem1)
        osem = (osem0, osem1)
        wid = lax.axis_index("s") * NC + lax.axis_index("c")
        base = wid * RPW

        pltpu.sync_copy(ids_hbm.at[wid], idx_v)

        def start_in(k, slot):
            pltpu.async_copy(table_hbm.at[idx_v.at[k]], tok[slot], gsem[slot])

        def wait_in(k, slot):
            pltpu.make_async_copy(
                table_hbm.at[idx_v.at[k]], tok[slot], gsem[slot]).wait()

        def start_out(k, slot):
            pltpu.async_copy(
                tok[slot], out_hbm.at[pl.ds(base + k * K, K)], osem[slot])

        def wait_out(k, slot):
            pltpu.make_async_copy(
                tok[slot], out_hbm.at[pl.ds(base + k * K, K)],
                osem[slot]).wait()

        def phase(k, cur, other):
            @pl.when(k >= 1)
            def _():
                wait_out(k - 1, other)

            @pl.when(k + 1 < NCH)
            def _():
                start_in(k + 1, other)

            wait_in(k, cur)
            start_out(k, cur)

        start_in(0, 0)

        def pair_body(c2, carry):
            phase(2 * c2, 0, 1)
            phase(2 * c2 + 1, 1, 0)
            return carry

        lax.fori_loop(0, NCH // 2, pair_body, 0)
        wait_out(NCH - 1, 1)

    return sc_gather


def _tc_ln_body(BLK, x_ref, pos_ref, g_ref, b_ref, o_ref):
    j = pl.program_id(0)
    x = x_ref[...] + pos_ref[pl.ds(j * BLK, BLK), :]
    mu = jnp.mean(x, axis=-1, keepdims=True)
    d = x - mu
    var = jnp.mean(d * d, axis=-1, keepdims=True)
    o_ref[...] = d * lax.rsqrt(var + 1e-5) * g_ref[...] + b_ref[...]


def _tc_ln_body_acc(BLK, acc_ref, x_ref, pos_ref, g_ref, b_ref, o_ref):
    _tc_ln_body(BLK, x_ref, pos_ref, g_ref, b_ref, o_ref)


@functools.cache
def _make_tc_ln(BS, S, D, HB, hoff, aliased):
    BLK = 1024
    nsb = S // BLK

    x_spec = pl.BlockSpec((BLK, D), lambda j, b: (b * nsb + j, 0))
    pos_spec = pl.BlockSpec((S, D), lambda j, b: (0, 0))
    vec_spec = pl.BlockSpec((1, D), lambda j, b: (0, 0))
    out_spec = pl.BlockSpec(
        (BLK, D), lambda j, b: ((hoff + b) * nsb + j, 0))
    if aliased:
        dummy = pl.BlockSpec((8, 128), lambda j, b: (0, 0))
        return pl.pallas_call(
            functools.partial(_tc_ln_body_acc, BLK),
            grid=(nsb, HB),
            in_specs=[dummy, x_spec, pos_spec, vec_spec, vec_spec],
            out_specs=out_spec,
            out_shape=jax.ShapeDtypeStruct((BS, D), jnp.float32),
            input_output_aliases={0: 0},
        )
    return pl.pallas_call(
        functools.partial(_tc_ln_body, BLK),
        grid=(nsb, HB),
        in_specs=[x_spec, pos_spec, vec_spec, vec_spec],
        out_specs=out_spec,
        out_shape=jax.ShapeDtypeStruct((BS, D), jnp.float32),
    )


def kernel(input_ids, token_table, pos_table, ln_gamma, ln_beta):
    B, S = input_ids.shape
    V, D = token_table.shape
    info = plsc.get_sparse_core_info()
    NC, NS = info.num_cores, info.num_subcores
    NW = NC * NS
    BS = B * S
    K = 32
    NSPLIT = 2
    HB = B // NSPLIT
    HR = BS // NSPLIT
    ids_flat = input_ids.astype(jnp.int32).reshape(BS)
    sc = _make_sc_gather(HR, V, D, NC, NS)
    g1 = ln_gamma.reshape(1, D)
    b1 = ln_beta.reshape(1, D)
    gathered = []
    for h in range(NSPLIT):
        idsh = lax.slice(ids_flat, (h * HR,), ((h + 1) * HR,))
        ids3 = idsh.reshape(NW, HR // NW // K, K)
        gathered.append(sc(ids3, token_table))
    out = _make_tc_ln(BS, S, D, HB, B - HB, False)(
        gathered[NSPLIT - 1], pos_table, g1, b1)
    for h in range(NSPLIT - 2, -1, -1):
        out = _make_tc_ln(BS, S, D, HB, h * HB, True)(
            out, gathered[h], pos_table, g1, b1)
    return out.reshape(B, S, D)


# speedup vs baseline: 1.4783x; 1.0609x over previous
import functools

import jax
import jax.numpy as jnp
from jax import lax
from jax.experimental import pallas as pl
from jax.experimental.pallas import tpu as pltpu
from jax.experimental.pallas import tpu_sc as plsc


@functools.cache
def _make_sc_gather(BS, V, D, NC, NS):
    NW = NC * NS
    RPW = BS // NW
    K = 32
    NCH = RPW // K
    mesh = plsc.VectorSubcoreMesh(core_axis_name="c", subcore_axis_name="s")

    @functools.partial(
        pl.kernel,
        mesh=mesh,
        out_type=jax.ShapeDtypeStruct((BS, D), jnp.float32),
        scratch_types=[
            pltpu.VMEM((NCH, K), jnp.int32),
            pltpu.VMEM((K, D), jnp.float32),
            pltpu.VMEM((K, D), jnp.float32),
            pltpu.SemaphoreType.DMA,
            pltpu.SemaphoreType.DMA,
            pltpu.SemaphoreType.DMA,
            pltpu.SemaphoreType.DMA,
        ],
    )
    def sc_gather(ids_hbm, table_hbm, out_hbm,
                  idx_v, tok0_v, tok1_v, gsem0, gsem1, osem0, osem1):
        tok = (tok0_v, tok1_v)
        gsem = (gsem0, gsem1)
        osem = (osem0, osem1)
        wid = lax.axis_index("s") * NC + lax.axis_index("c")
        base = wid * RPW

        pltpu.sync_copy(ids_hbm.at[wid], idx_v)

        def start_in(k, slot):
            pltpu.async_copy(table_hbm.at[idx_v.at[k]], tok[slot], gsem[slot])

        def wait_in(k, slot):
            pltpu.make_async_copy(
                table_hbm.at[idx_v.at[k]], tok[slot], gsem[slot]).wait()

        def start_out(k, slot):
            pltpu.async_copy(
                tok[slot], out_hbm.at[pl.ds(base + k * K, K)], osem[slot])

        def wait_out(k, slot):
            pltpu.make_async_copy(
                tok[slot], out_hbm.at[pl.ds(base + k * K, K)],
                osem[slot]).wait()

        def phase(k, cur, other):
            @pl.when(k >= 1)
            def _():
                wait_out(k - 1, other)

            @pl.when(k + 1 < NCH)
            def _():
                start_in(k + 1, other)

            wait_in(k, cur)
            start_out(k, cur)

        start_in(0, 0)

        def pair_body(c2, carry):
            phase(2 * c2, 0, 1)
            phase(2 * c2 + 1, 1, 0)
            return carry

        lax.fori_loop(0, NCH // 2, pair_body, 0)
        wait_out(NCH - 1, 1)

    return sc_gather


def _tc_ln_body(BLK, x_ref, pos_ref, g_ref, b_ref, o_ref):
    j = pl.program_id(0)
    x = x_ref[...] + pos_ref[pl.ds(j * BLK, BLK), :]
    mu = jnp.mean(x, axis=-1, keepdims=True)
    d = x - mu
    var = jnp.mean(d * d, axis=-1, keepdims=True)
    o_ref[...] = d * lax.rsqrt(var + 1e-5) * g_ref[...] + b_ref[...]


def _tc_ln_body_acc(BLK, acc_ref, x_ref, pos_ref, g_ref, b_ref, o_ref):
    _tc_ln_body(BLK, x_ref, pos_ref, g_ref, b_ref, o_ref)


@functools.cache
def _make_tc_ln(BS, S, D, HB, hoff, aliased):
    BLK = 1024
    nsb = S // BLK

    x_spec = pl.BlockSpec((BLK, D), lambda j, b: (b * nsb + j, 0))
    pos_spec = pl.BlockSpec((S, D), lambda j, b: (0, 0))
    vec_spec = pl.BlockSpec((1, D), lambda j, b: (0, 0))
    out_spec = pl.BlockSpec(
        (BLK, D), lambda j, b: ((hoff + b) * nsb + j, 0))
    if aliased:
        dummy = pl.BlockSpec((8, 128), lambda j, b: (0, 0))
        return pl.pallas_call(
            functools.partial(_tc_ln_body_acc, BLK),
            grid=(nsb, HB),
            in_specs=[dummy, x_spec, pos_spec, vec_spec, vec_spec],
            out_specs=out_spec,
            out_shape=jax.ShapeDtypeStruct((BS, D), jnp.float32),
            input_output_aliases={0: 0},
        )
    return pl.pallas_call(
        functools.partial(_tc_ln_body, BLK),
        grid=(nsb, HB),
        in_specs=[x_spec, pos_spec, vec_spec, vec_spec],
        out_specs=out_spec,
        out_shape=jax.ShapeDtypeStruct((BS, D), jnp.float32),
    )


def kernel(input_ids, token_table, pos_table, ln_gamma, ln_beta):
    B, S = input_ids.shape
    V, D = token_table.shape
    info = plsc.get_sparse_core_info()
    NC, NS = info.num_cores, info.num_subcores
    NW = NC * NS
    BS = B * S
    K = 32
    NSPLIT = 1
    HB = B // NSPLIT
    HR = BS // NSPLIT
    ids_flat = input_ids.astype(jnp.int32).reshape(BS)
    sc = _make_sc_gather(HR, V, D, NC, NS)
    g1 = ln_gamma.reshape(1, D)
    b1 = ln_beta.reshape(1, D)
    gathered = []
    for h in range(NSPLIT):
        idsh = lax.slice(ids_flat, (h * HR,), ((h + 1) * HR,))
        ids3 = idsh.reshape(NW, HR // NW // K, K)
        gathered.append(sc(ids3, token_table))
    out = _make_tc_ln(BS, S, D, HB, B - HB, False)(
        gathered[NSPLIT - 1], pos_table, g1, b1)
    for h in range(NSPLIT - 2, -1, -1):
        out = _make_tc_ln(BS, S, D, HB, h * HB, True)(
            out, gathered[h], pos_table, g1, b1)
    return out.reshape(B, S, D)
